# baseline (speedup 1.0000x reference)
import functools

import jax
import jax.numpy as jnp
from jax import lax
from jax.experimental import pallas as pl
from jax.experimental.pallas import tpu as pltpu

_NUM_CLASSES = 21
_KNN = 5
_CONCENTRATION = 16.0
_LAB_BITS = 5


def _topk_vote(sim, key, bq):
    big = jnp.float32(3e38)
    neg = jnp.float32(-jnp.inf)
    cls = lax.broadcasted_iota(jnp.int32, (1, 32), 1)
    scores = jnp.zeros((bq, 32), jnp.float32)
    work = sim
    k = sim.shape[1]
    cch = 128
    wch = k // cch
    mch = jnp.max(work.reshape(bq, cch, wch), axis=2)
    for it in range(_KNN):
        m = jnp.max(mch, axis=1, keepdims=True)
        amin = jnp.min(jnp.where(work == m, key, big),
                       axis=1, keepdims=True)
        lab = amin.astype(jnp.int32) & (2 ** _LAB_BITS - 1)
        scores = scores + m * (lab == cls).astype(jnp.float32)
        if it < _KNN - 1:
            work = jnp.where(key == amin, neg, work)
            mch = jnp.max(work.reshape(bq, cch, wch), axis=2)
    return scores


def _body(e_ref, p_ref, lab_ref, o_ref, pn_ref, *, bq, k, d):
    @pl.when(pl.program_id(0) == 0)
    def _():
        p = p_ref[...]
        pnorm = jnp.sqrt(jnp.sum(p * p, axis=1, keepdims=True)) + 1e-12
        pn_ref[...] = p / pnorm

    e = e_ref[...]
    scale = _CONCENTRATION / (
        jnp.sqrt(jnp.sum(e * e, axis=1, keepdims=True)) + 1e-12)
    sim = lax.dot_general(
        e * scale, pn_ref[...], (((1,), (1,)), ((), ())),
        preferred_element_type=jnp.float32,
    )

    col = lax.broadcasted_iota(jnp.int32, (1, k), 1)
    key = ((col << _LAB_BITS) | lab_ref[...]).astype(jnp.float32)
    scores = _topk_vote(sim, key, bq)
    o_ref[...] = scores[:, :_NUM_CLASSES]


def kernel(embeddings, prototypes, prototype_labels):
    q, d = embeddings.shape
    k = prototypes.shape[0]
    bq = 256
    labels2d = prototype_labels.reshape(1, k)

    return pl.pallas_call(
        functools.partial(_body, bq=bq, k=k, d=d),
        grid=(q // bq,),
        in_specs=[
            pl.BlockSpec((bq, d), lambda i: (i, 0)),
            pl.BlockSpec((k, d), lambda i: (0, 0)),
            pl.BlockSpec((1, k), lambda i: (0, 0)),
        ],
        out_specs=pl.BlockSpec((bq, _NUM_CLASSES), lambda i: (i, 0)),
        out_shape=jax.ShapeDtypeStruct((q, _NUM_CLASSES), jnp.float32),
        scratch_shapes=[pltpu.VMEM((k, d), jnp.float32)],
    )(embeddings, prototypes, labels2d)


# speedup vs baseline: 2.7305x; 2.7305x over previous
import functools

import jax
import jax.numpy as jnp
from jax import lax
from jax.experimental import pallas as pl
from jax.experimental.pallas import tpu as pltpu
from jax.experimental.pallas import tpu_sc as plsc

_NUM_CLASSES = 21
_KNN = 5
_CONCENTRATION = 16.0
_LAB_BITS = 5


def _topk(sim, key, bq):
    big = jnp.float32(3e38)
    neg = jnp.float32(-jnp.inf)
    c8 = lax.broadcasted_iota(jnp.int32, (1, 8), 1)
    vals8 = jnp.zeros((bq, 8), jnp.float32)
    idx8 = jnp.zeros((bq, 8), jnp.int32)
    work = sim
    for it in range(_KNN):
        m = jnp.max(work, axis=1, keepdims=True)
        amin = jnp.min(jnp.where(work == m, key, big),
                       axis=1, keepdims=True)
        sel = c8 == it
        vals8 = jnp.where(sel, m, vals8)
        idx8 = jnp.where(sel, amin.astype(jnp.int32) >> _LAB_BITS, idx8)
        if it < _KNN - 1:
            work = jnp.where(key == amin, neg, work)
    return vals8, idx8


def _body(e_ref, p_ref, lab_ref, ov_ref, oi_ref, pn_ref, *, bq, k, d):
    @pl.when(pl.program_id(0) == 0)
    def _():
        p = p_ref[...]
        pnorm = jnp.sqrt(jnp.sum(p * p, axis=1, keepdims=True)) + 1e-12
        pn_ref[...] = p / pnorm

    e = e_ref[...]
    scale = _CONCENTRATION / (
        jnp.sqrt(jnp.sum(e * e, axis=1, keepdims=True)) + 1e-12)
    sim = lax.dot_general(
        e * scale, pn_ref[...], (((1,), (1,)), ((), ())),
        preferred_element_type=jnp.float32,
    )

    col = lax.broadcasted_iota(jnp.int32, (1, k), 1)
    key = ((col << _LAB_BITS) | lab_ref[...]).astype(jnp.float32)
    vals8, idx8 = _topk(sim, key, bq)
    ov_ref[...] = vals8
    oi_ref[...] = idx8


def _make_sc_vote(q, k):
    nw = 32
    rpw = q // nw
    mesh = plsc.VectorSubcoreMesh(core_axis_name="c", subcore_axis_name="s")

    @functools.partial(
        pl.kernel, mesh=mesh,
        out_type=jax.ShapeDtypeStruct((q, 32), jnp.float32),
        compiler_params=pltpu.CompilerParams(needs_layout_passes=False),
        scratch_types=[
            pltpu.VMEM((rpw * 8,), jnp.int32),
            pltpu.VMEM((rpw * 8,), jnp.float32),
            pltpu.VMEM((k,), jnp.int32),
            pltpu.VMEM((rpw, 32), jnp.float32),
        ],
    )
    def sc_vote(lab_hbm, idx_hbm, val_hbm, out_hbm, idxv, valv, labv, acc):
        wid = lax.axis_index("s") * 2 + lax.axis_index("c")
        base = wid * (rpw * 8)
        pltpu.sync_copy(lab_hbm, labv)
        pltpu.sync_copy(idx_hbm.at[pl.ds(base, rpw * 8)], idxv)
        pltpu.sync_copy(val_hbm.at[pl.ds(base, rpw * 8)], valv)
        zero = jnp.zeros((16,), jnp.float32)
        for r in range(rpw):
            acc[r, pl.ds(0, 16)] = zero
            acc[r, pl.ds(16, 16)] = zero
        rbase = jnp.arange(16, dtype=jnp.int32)
        for g in range(rpw // 16):
            rvec = rbase + g * 16
            for t in range(_KNN):
                pos = rvec * 8 + t
                iv = plsc.load_gather(idxv, [pos])
                lab = plsc.load_gather(labv, [iv])
                v = plsc.load_gather(valv, [pos])
                plsc.addupdate_scatter(acc, [rvec, lab], v)
        pltpu.sync_copy(acc, out_hbm.at[pl.ds(wid * rpw, rpw)])

    return sc_vote


def kernel(embeddings, prototypes, prototype_labels):
    q, d = embeddings.shape
    k = prototypes.shape[0]
    bq = 256
    labels2d = prototype_labels.reshape(1, k)

    vals8, idx8 = pl.pallas_call(
        functools.partial(_body, bq=bq, k=k, d=d),
        grid=(q // bq,),
        in_specs=[
            pl.BlockSpec((bq, d), lambda i: (i, 0)),
            pl.BlockSpec((k, d), lambda i: (0, 0)),
            pl.BlockSpec((1, k), lambda i: (0, 0)),
        ],
        out_specs=[
            pl.BlockSpec((bq, 8), lambda i: (i, 0)),
            pl.BlockSpec((bq, 8), lambda i: (i, 0)),
        ],
        out_shape=[
            jax.ShapeDtypeStruct((q, 8), jnp.float32),
            jax.ShapeDtypeStruct((q, 8), jnp.int32),
        ],
        scratch_shapes=[pltpu.VMEM((k, d), jnp.float32)],
    )(embeddings, prototypes, labels2d)

    sc_vote = _make_sc_vote(q, k)
    scores = sc_vote(prototype_labels, idx8.reshape(-1), vals8.reshape(-1))
    return scores[:, :_NUM_CLASSES]


# speedup vs baseline: 2.9767x; 1.0901x over previous
import functools

import jax
import jax.numpy as jnp
from jax import lax
from jax.experimental import pallas as pl
from jax.experimental.pallas import tpu as pltpu

_NUM_CLASSES = 21
_KNN = 5
_CONCENTRATION = 16.0
_LAB_BITS = 5


def _topk_vote(sim, key, bq):
    big = jnp.float32(3e38)
    neg = jnp.float32(-jnp.inf)
    cls = lax.broadcasted_iota(jnp.int32, (1, 32), 1)
    scores = jnp.zeros((bq, 32), jnp.float32)
    work = sim
    for it in range(_KNN):
        m = jnp.max(work, axis=1, keepdims=True)
        amin = jnp.min(jnp.where(work == m, key, big),
                       axis=1, keepdims=True)
        lab = amin.astype(jnp.int32) & (2 ** _LAB_BITS - 1)
        scores = scores + m * (lab == cls).astype(jnp.float32)
        if it < _KNN - 1:
            work = jnp.where(key == amin, neg, work)
    return scores


def _body(e_ref, p_ref, lab_ref, o_ref, pn_ref, *, bq, k, d):
    @pl.when(pl.program_id(0) == 0)
    def _():
        p = p_ref[...]
        pnorm = jnp.sqrt(jnp.sum(p * p, axis=1, keepdims=True)) + 1e-12
        pn_ref[...] = p / pnorm

    e = e_ref[...]
    scale = _CONCENTRATION / (
        jnp.sqrt(jnp.sum(e * e, axis=1, keepdims=True)) + 1e-12)
    sim = lax.dot_general(
        e * scale, pn_ref[...], (((1,), (1,)), ((), ())),
        preferred_element_type=jnp.float32,
    )

    col = lax.broadcasted_iota(jnp.int32, (1, k), 1)
    key = ((col << _LAB_BITS) | lab_ref[...]).astype(jnp.float32)
    scores = _topk_vote(sim, key, bq)
    o_ref[...] = scores[:, :_NUM_CLASSES]


def kernel(embeddings, prototypes, prototype_labels):
    q, d = embeddings.shape
    k = prototypes.shape[0]
    bq = 256
    labels2d = prototype_labels.reshape(1, k)

    return pl.pallas_call(
        functools.partial(_body, bq=bq, k=k, d=d),
        grid=(q // bq,),
        in_specs=[
            pl.BlockSpec((bq, d), lambda i: (i, 0)),
            pl.BlockSpec((k, d), lambda i: (0, 0)),
            pl.BlockSpec((1, k), lambda i: (0, 0)),
        ],
        out_specs=pl.BlockSpec((bq, _NUM_CLASSES), lambda i: (i, 0)),
        out_shape=jax.ShapeDtypeStruct((q, _NUM_CLASSES), jnp.float32),
        scratch_shapes=[pltpu.VMEM((k, d), jnp.float32)],
    )(embeddings, prototypes, labels2d)
